# baseline (speedup 1.0000x reference)
import functools

import jax
import jax.numpy as jnp
from jax import lax
from jax.experimental import pallas as pl
from jax.experimental.pallas import tpu as pltpu
from jax.experimental.pallas import tpu_sc as plsc

_VOCAB = 100000
_EMBED_DIM = 128
_SEQ_LEN = 200
_BATCH = 4096

_NUM_WORKERS = 32
_B_PER_W = _BATCH // _NUM_WORKERS
_LANES = 16
_INV_L = 1.0 / _SEQ_LEN


def _tw_body(table_ref, w_ref, bias_ref, out_ref):
    acc = jnp.sum(table_ref[...] * w_ref[...], axis=1, keepdims=True)
    out_ref[...] = acc * _INV_L + bias_ref[0] * _INV_L


def _compute_tw(table, w, bias):
    blk = 10000
    grid = _VOCAB // blk
    return pl.pallas_call(
        _tw_body,
        grid=(grid,),
        in_specs=[
            pl.BlockSpec((blk, _EMBED_DIM), lambda i: (i, 0)),
            pl.BlockSpec((1, _EMBED_DIM), lambda i: (0, 0)),
            pl.BlockSpec(memory_space=pltpu.SMEM),
        ],
        out_specs=pl.BlockSpec((blk, 1), lambda i: (i, 0)),
        out_shape=jax.ShapeDtypeStruct((_VOCAB, 1), jnp.float32),
    )(table, w, bias)


def _make_sc_gather_sum():
    mesh = plsc.VectorSubcoreMesh(core_axis_name="c", subcore_axis_name="s")

    @functools.partial(
        pl.kernel,
        mesh=mesh,
        compiler_params=pltpu.CompilerParams(needs_layout_passes=False),
        out_type=jax.ShapeDtypeStruct((_BATCH,), jnp.float32),
        scratch_types=[
            pltpu.VMEM((_VOCAB,), jnp.float32),
            pltpu.VMEM((_SEQ_LEN, _B_PER_W), jnp.int32),
            pltpu.VMEM((_B_PER_W,), jnp.float32),
            pltpu.SemaphoreType.DMA,
            pltpu.SemaphoreType.DMA,
        ],
    )
    def sc_kernel(tw_hbm, x_hbm, out_hbm, tw_v, idx_v, out_v, sem_tw, sem_x):
        wid = lax.axis_index("s") * 2 + lax.axis_index("c")
        base = wid * _B_PER_W
        cp_tw = pltpu.async_copy(tw_hbm, tw_v, sem_tw)
        cp_x = pltpu.async_copy(x_hbm.at[:, pl.ds(base, _B_PER_W)], idx_v,
                                sem_x)
        cp_tw.wait()
        cp_x.wait()

        nv = _B_PER_W // _LANES

        def body(l, accs):
            new = []
            for j in range(nv):
                idx = idx_v[l, pl.ds(j * _LANES, _LANES)]
                new.append(accs[j] + plsc.load_gather(tw_v, [idx]))
            return tuple(new)

        accs = lax.fori_loop(
            0, _SEQ_LEN, body,
            tuple(jnp.zeros((_LANES,), jnp.float32) for _ in range(nv)))
        for j in range(nv):
            out_v[pl.ds(j * _LANES, _LANES)] = accs[j]
        pltpu.sync_copy(out_v, out_hbm.at[pl.ds(base, _B_PER_W)])

    return sc_kernel


_sc_gather_sum = _make_sc_gather_sum()


def kernel(x, table, W, b):
    tw = _compute_tw(table, W, b)
    return _sc_gather_sum(tw.reshape(_VOCAB), x)


# speedup vs baseline: 1.0053x; 1.0053x over previous
import functools

import jax
import jax.numpy as jnp
from jax import lax
from jax.experimental import pallas as pl
from jax.experimental.pallas import tpu as pltpu
from jax.experimental.pallas import tpu_sc as plsc

_VOCAB = 100000
_EMBED_DIM = 128
_SEQ_LEN = 200
_BATCH = 4096

_NUM_WORKERS = 32
_B_PER_W = _BATCH // _NUM_WORKERS
_LANES = 16
_INV_L = 1.0 / _SEQ_LEN


def _tw_body(table_ref, w_ref, bias_ref, out_ref):
    acc = jnp.sum(table_ref[...] * w_ref[...], axis=1, keepdims=True)
    out_ref[...] = acc * _INV_L + bias_ref[0] * _INV_L


def _compute_tw(table, w, bias):
    blk = 20000
    grid = _VOCAB // blk
    return pl.pallas_call(
        _tw_body,
        grid=(grid,),
        in_specs=[
            pl.BlockSpec((blk, _EMBED_DIM), lambda i: (i, 0)),
            pl.BlockSpec((1, _EMBED_DIM), lambda i: (0, 0)),
            pl.BlockSpec(memory_space=pltpu.SMEM),
        ],
        out_specs=pl.BlockSpec((blk, 1), lambda i: (i, 0)),
        out_shape=jax.ShapeDtypeStruct((_VOCAB, 1), jnp.float32),
    )(table, w, bias)


def _make_sc_gather_sum():
    mesh = plsc.VectorSubcoreMesh(core_axis_name="c", subcore_axis_name="s")

    @functools.partial(
        pl.kernel,
        mesh=mesh,
        compiler_params=pltpu.CompilerParams(needs_layout_passes=False),
        out_type=jax.ShapeDtypeStruct((_BATCH,), jnp.float32),
        scratch_types=[
            pltpu.VMEM((_VOCAB,), jnp.float32),
            pltpu.VMEM((_SEQ_LEN, _B_PER_W), jnp.int32),
            pltpu.VMEM((_B_PER_W,), jnp.float32),
            pltpu.SemaphoreType.DMA,
            pltpu.SemaphoreType.DMA,
        ],
    )
    def sc_kernel(tw_hbm, x_hbm, out_hbm, tw_v, idx_v, out_v, sem_tw, sem_x):
        wid = lax.axis_index("s") * 2 + lax.axis_index("c")
        base = wid * _B_PER_W
        cp_tw = pltpu.async_copy(tw_hbm, tw_v, sem_tw)
        cp_x = pltpu.async_copy(x_hbm.at[:, pl.ds(base, _B_PER_W)], idx_v,
                                sem_x)
        cp_tw.wait()
        cp_x.wait()

        nv = _B_PER_W // _LANES

        def body(l, accs):
            new = []
            for j in range(nv):
                idx = idx_v[l, pl.ds(j * _LANES, _LANES)]
                new.append(accs[j] + plsc.load_gather(tw_v, [idx]))
            return tuple(new)

        accs = lax.fori_loop(
            0, _SEQ_LEN, body,
            tuple(jnp.zeros((_LANES,), jnp.float32) for _ in range(nv)))
        for j in range(nv):
            out_v[pl.ds(j * _LANES, _LANES)] = accs[j]
        pltpu.sync_copy(out_v, out_hbm.at[pl.ds(base, _B_PER_W)])

    return sc_kernel


_sc_gather_sum = _make_sc_gather_sum()


def kernel(x, table, W, b):
    tw = _compute_tw(table, W, b)
    return _sc_gather_sum(tw.reshape(_VOCAB), x)


# speedup vs baseline: 1.0803x; 1.0745x over previous
import functools

import jax
import jax.numpy as jnp
from jax import lax
from jax.experimental import pallas as pl
from jax.experimental.pallas import tpu as pltpu
from jax.experimental.pallas import tpu_sc as plsc

_VOCAB = 100000
_EMBED_DIM = 128
_SEQ_LEN = 200
_BATCH = 4096

_NUM_WORKERS = 32
_B_PER_W = _BATCH // _NUM_WORKERS
_LANES = 16
_INV_L = 1.0 / _SEQ_LEN


_VOCAB_PAD = 100352


def _tw_body(table_ref, w_ref, bias_ref, out_ref):
    acc = jnp.sum(table_ref[...] * w_ref[...], axis=1)
    out_ref[...] = acc * _INV_L + bias_ref[0] * _INV_L


def _compute_tw(table, w, bias):
    blk = _VOCAB_PAD // 7
    return pl.pallas_call(
        _tw_body,
        grid=(7,),
        in_specs=[
            pl.BlockSpec((blk, _EMBED_DIM), lambda i: (i, 0)),
            pl.BlockSpec((1, _EMBED_DIM), lambda i: (0, 0)),
            pl.BlockSpec(memory_space=pltpu.SMEM),
        ],
        out_specs=pl.BlockSpec((blk,), lambda i: (i,)),
        out_shape=jax.ShapeDtypeStruct((_VOCAB_PAD,), jnp.float32),
    )(table, w, bias)


def _make_sc_gather_sum():
    mesh = plsc.VectorSubcoreMesh(core_axis_name="c", subcore_axis_name="s")

    @functools.partial(
        pl.kernel,
        mesh=mesh,
        compiler_params=pltpu.CompilerParams(needs_layout_passes=False),
        out_type=jax.ShapeDtypeStruct((_BATCH,), jnp.float32),
        scratch_types=[
            pltpu.VMEM((_VOCAB_PAD,), jnp.float32),
            pltpu.VMEM((_SEQ_LEN, _B_PER_W), jnp.int32),
            pltpu.VMEM((_B_PER_W,), jnp.float32),
            pltpu.SemaphoreType.DMA,
            pltpu.SemaphoreType.DMA,
        ],
    )
    def sc_kernel(tw_hbm, x_hbm, out_hbm, tw_v, idx_v, out_v, sem_tw, sem_x):
        wid = lax.axis_index("s") * 2 + lax.axis_index("c")
        base = wid * _B_PER_W
        cp_tw = pltpu.async_copy(tw_hbm, tw_v, sem_tw)
        cp_x = pltpu.async_copy(x_hbm.at[:, pl.ds(base, _B_PER_W)], idx_v,
                                sem_x)
        cp_tw.wait()
        cp_x.wait()

        nv = _B_PER_W // _LANES

        def body(l, accs):
            new = []
            for j in range(nv):
                idx = idx_v[l, pl.ds(j * _LANES, _LANES)]
                new.append(accs[j] + plsc.load_gather(tw_v, [idx]))
            return tuple(new)

        accs = lax.fori_loop(
            0, _SEQ_LEN, body,
            tuple(jnp.zeros((_LANES,), jnp.float32) for _ in range(nv)))
        for j in range(nv):
            out_v[pl.ds(j * _LANES, _LANES)] = accs[j]
        pltpu.sync_copy(out_v, out_hbm.at[pl.ds(base, _B_PER_W)])

    return sc_kernel


_sc_gather_sum = _make_sc_gather_sum()


def kernel(x, table, W, b):
    tw = _compute_tw(table, W, b)
    return _sc_gather_sum(tw, x)


# speedup vs baseline: 1.7555x; 1.6250x over previous
import functools

import jax
import jax.numpy as jnp
from jax import lax
from jax.experimental import pallas as pl
from jax.experimental.pallas import tpu as pltpu
from jax.experimental.pallas import tpu_sc as plsc

_VOCAB = 100000
_EMBED_DIM = 128
_SEQ_LEN = 200
_BATCH = 4096

_NUM_WORKERS = 32
_B_PER_W = _BATCH // _NUM_WORKERS
_LANES = 16
_INV_L = 1.0 / _SEQ_LEN


_VOCAB_PAD = 100352


def _tw_body(table_ref, w_ref, bias_ref, out_ref):
    acc = lax.dot_general(
        w_ref[...], table_ref[...],
        dimension_numbers=(((1,), (1,)), ((), ())),
        preferred_element_type=jnp.float32)
    out_ref[...] = acc[0] * _INV_L + bias_ref[0] * _INV_L


def _compute_tw(table, w, bias):
    blk = _VOCAB_PAD // 7
    return pl.pallas_call(
        _tw_body,
        grid=(7,),
        in_specs=[
            pl.BlockSpec((blk, _EMBED_DIM), lambda i: (i, 0)),
            pl.BlockSpec((1, _EMBED_DIM), lambda i: (0, 0)),
            pl.BlockSpec(memory_space=pltpu.SMEM),
        ],
        out_specs=pl.BlockSpec((blk,), lambda i: (i,)),
        out_shape=jax.ShapeDtypeStruct((_VOCAB_PAD,), jnp.float32),
    )(table, w, bias)


def _make_sc_gather_sum():
    mesh = plsc.VectorSubcoreMesh(core_axis_name="c", subcore_axis_name="s")

    @functools.partial(
        pl.kernel,
        mesh=mesh,
        compiler_params=pltpu.CompilerParams(needs_layout_passes=False),
        out_type=jax.ShapeDtypeStruct((_BATCH,), jnp.float32),
        scratch_types=[
            pltpu.VMEM((_VOCAB_PAD,), jnp.float32),
            pltpu.VMEM((_SEQ_LEN, _B_PER_W), jnp.int32),
            pltpu.VMEM((_B_PER_W,), jnp.float32),
            pltpu.SemaphoreType.DMA,
            pltpu.SemaphoreType.DMA,
        ],
    )
    def sc_kernel(tw_hbm, x_hbm, out_hbm, tw_v, idx_v, out_v, sem_tw, sem_x):
        wid = lax.axis_index("s") * 2 + lax.axis_index("c")
        base = wid * _B_PER_W
        cp_tw = pltpu.async_copy(tw_hbm, tw_v, sem_tw)
        cp_x = pltpu.async_copy(x_hbm.at[:, pl.ds(base, _B_PER_W)], idx_v,
                                sem_x)
        cp_tw.wait()
        cp_x.wait()

        nv = _B_PER_W // _LANES

        def body(l, accs):
            new = []
            for j in range(nv):
                idx = idx_v[l, pl.ds(j * _LANES, _LANES)]
                new.append(accs[j] + plsc.load_gather(tw_v, [idx]))
            return tuple(new)

        accs = lax.fori_loop(
            0, _SEQ_LEN, body,
            tuple(jnp.zeros((_LANES,), jnp.float32) for _ in range(nv)))
        for j in range(nv):
            out_v[pl.ds(j * _LANES, _LANES)] = accs[j]
        pltpu.sync_copy(out_v, out_hbm.at[pl.ds(base, _B_PER_W)])

    return sc_kernel


_sc_gather_sum = _make_sc_gather_sum()


def kernel(x, table, W, b):
    tw = _compute_tw(table, W, b)
    return _sc_gather_sum(tw, x)
